# baseline (speedup 1.0000x reference)
import functools

import jax
import jax.numpy as jnp
from jax import lax
from jax.experimental import layout as jex_layout
from jax.experimental import pallas as pl
from jax.experimental.pallas import tpu as pltpu
from jax.experimental.pallas import tpu_sc as plsc

_NC = 2
_NS = 16
_NW = _NC * _NS
_L = 16


def _gather_pool(B, S, D, CB=4):
    b_per_w = B // _NW
    ROWS = CB * S
    NCHUNK = b_per_w // CB
    assert B % _NW == 0 and b_per_w % CB == 0 and ROWS % 8 == 0
    assert D == 2 * _L and S % 4 == 0

    mesh = plsc.VectorSubcoreMesh(core_axis_name="c", subcore_axis_name="s",
                                  num_cores=_NC, num_subcores=_NS)

    @functools.partial(
        pl.kernel,
        out_type=jax.ShapeDtypeStruct((B, D), jnp.float32),
        mesh=mesh,
        scratch_types=[
            pltpu.VMEM((4 * ROWS,), jnp.int32),
            pltpu.VMEM((2, ROWS, D), jnp.bfloat16),
            pltpu.VMEM((b_per_w, D), jnp.float32),
            pltpu.SemaphoreType.DMA,
            pltpu.SemaphoreType.DMA,
            pltpu.SemaphoreType.DMA,
            pltpu.SemaphoreType.DMA,
            pltpu.SemaphoreType.DMA,
            pltpu.SemaphoreType.DMA,
        ],
        compiler_params=pltpu.CompilerParams(use_tc_tiling_on_sc=False,
                                             needs_layout_passes=False),
    )
    def kern(x_hbm, table_hbm, out_hbm, idx_v, rows_v, out_v,
             si0, si1, si2, si3, sr0, sr1):
        si = (si0, si1, si2, si3)
        sr = (sr0, sr1)
        wid = lax.axis_index("s") * _NC + lax.axis_index("c")
        xbase = wid * (b_per_w * S)

        def islot(slot):
            return idx_v.at[pl.ds(slot * ROWS, ROWS)]

        def start_idx(c, slot):
            pltpu.async_copy(
                x_hbm.at[pl.ds(xbase + c * ROWS, ROWS)], islot(slot),
                si[slot])

        def wait_idx(slot):
            pltpu.make_async_copy(
                x_hbm.at[pl.ds(0, ROWS)], islot(slot), si[slot]).wait()

        def start_gather(idx_slot, row_slot):
            pltpu.async_copy(
                table_hbm.at[islot(idx_slot)], rows_v.at[row_slot],
                sr[row_slot])

        def wait_gather(idx_slot, row_slot):
            pltpu.make_async_copy(
                table_hbm.at[islot(idx_slot)], rows_v.at[row_slot],
                sr[row_slot]).wait()

        def accum_chunk(row_slot, c):
            rv = rows_v.at[row_slot]
            seg = S // 4
            for r in range(CB):
                base = r * S
                zero = jnp.zeros((_L,), jnp.float32)

                @plsc.parallel_loop(0, seg, 1, unroll=4,
                                    carry=(zero,) * 8)
                def body(s, acc):
                    out = []
                    for g in range(4):
                        row = rv[base + g * seg + s, 0:D]
                        e, o = plsc.unpack(
                            row, format=plsc.PackFormat.INTERLEAVED)
                        out.extend((acc[2 * g] + e, acc[2 * g + 1] + o))
                    return tuple(out)

                acc = body
                h0 = (acc[0] + acc[2]) + (acc[4] + acc[6])
                h1 = (acc[1] + acc[3]) + (acc[5] + acc[7])
                out_v[c * CB + r, 0:_L] = h0
                out_v[c * CB + r, _L:D] = h1

        for k in range(4):
            start_idx(k, k)
        for k in range(2):
            wait_idx(k)
            start_gather(k, k)

        def step(g4, _):
            for b in range(4):
                c = 4 * g4 + b
                rslot = b % 2
                wait_gather(b, rslot)
                accum_chunk(rslot, c)

                @pl.when(c + 2 < NCHUNK)
                def _():
                    wait_idx((b + 2) % 4)
                    start_gather((b + 2) % 4, rslot)

                @pl.when(c + 4 < NCHUNK)
                def _():
                    start_idx(c + 4, b)
            return 0

        lax.fori_loop(0, NCHUNK // 4, step, 0)
        pltpu.sync_copy(out_v, out_hbm.at[pl.ds(wid * b_per_w, b_per_w)])

    return kern


def _linear(B, S, D, C, BT=2048):
    scale = 1.0 / S

    def body(p_ref, w_ref, b_ref, o_ref):
        acc = jnp.dot(p_ref[...], w_ref[...],
                      preferred_element_type=jnp.float32)
        o_ref[...] = acc * scale + b_ref[...]

    return pl.pallas_call(
        body,
        grid=(B // BT,),
        in_specs=[
            pl.BlockSpec((BT, D), lambda i: (i, 0)),
            pl.BlockSpec((D, C), lambda i: (0, 0)),
            pl.BlockSpec((1, C), lambda i: (0, 0)),
        ],
        out_specs=pl.BlockSpec((BT, C), lambda i: (i, 0)),
        out_shape=jax.ShapeDtypeStruct((B, C), jnp.float32),
    )


def kernel(x, table, W, b):
    B, S = x.shape
    V, D = table.shape
    C = W.shape[1]
    x_flat = x.reshape(-1).astype(jnp.int32)
    sums = _gather_pool(B, S, D)(x_flat, table.astype(jnp.bfloat16))
    W_perm = jnp.concatenate([W[0::2], W[1::2]], axis=0)
    return _linear(B, S, D, C)(sums, W_perm, b.reshape(1, C))


# speedup vs baseline: 1.1374x; 1.1374x over previous
import functools

import jax
import jax.numpy as jnp
from jax import lax
from jax.experimental import pallas as pl
from jax.experimental.pallas import tpu as pltpu
from jax.experimental.pallas import tpu_sc as plsc

_NC = 2
_NS = 16
_NW = _NC * _NS
_L = 16


def _gather_pool(B, S, D, CB=8):
    b_per_w = B // _NW
    ROWS = CB * S
    NCHUNK = b_per_w // CB
    assert B % _NW == 0 and b_per_w % CB == 0 and ROWS % 8 == 0
    assert D == 2 * _L and S % 4 == 0

    mesh = plsc.VectorSubcoreMesh(core_axis_name="c", subcore_axis_name="s",
                                  num_cores=_NC, num_subcores=_NS)

    @functools.partial(
        pl.kernel,
        out_type=jax.ShapeDtypeStruct((B, D), jnp.float32),
        mesh=mesh,
        scratch_types=[
            pltpu.VMEM((4 * ROWS,), jnp.int32),
            pltpu.VMEM((2, ROWS, D), jnp.float32),
            pltpu.VMEM((b_per_w, D), jnp.float32),
            pltpu.SemaphoreType.DMA,
            pltpu.SemaphoreType.DMA,
            pltpu.SemaphoreType.DMA,
            pltpu.SemaphoreType.DMA,
            pltpu.SemaphoreType.DMA,
            pltpu.SemaphoreType.DMA,
        ],
        compiler_params=pltpu.CompilerParams(use_tc_tiling_on_sc=False),
    )
    def kern(x_hbm, table_hbm, out_hbm, idx_v, rows_v, out_v,
             si0, si1, si2, si3, sr0, sr1):
        si = (si0, si1, si2, si3)
        sr = (sr0, sr1)
        wid = lax.axis_index("s") * _NC + lax.axis_index("c")
        xbase = wid * (b_per_w * S)

        def islot(slot):
            return idx_v.at[pl.ds(slot * ROWS, ROWS)]

        def start_idx(c, slot):
            pltpu.async_copy(
                x_hbm.at[pl.ds(xbase + c * ROWS, ROWS)], islot(slot),
                si[slot])

        def wait_idx(slot):
            pltpu.make_async_copy(
                x_hbm.at[pl.ds(0, ROWS)], islot(slot), si[slot]).wait()

        def start_gather(idx_slot, row_slot):
            pltpu.async_copy(
                table_hbm.at[islot(idx_slot)], rows_v.at[row_slot],
                sr[row_slot])

        def wait_gather(idx_slot, row_slot):
            pltpu.make_async_copy(
                table_hbm.at[islot(idx_slot)], rows_v.at[row_slot],
                sr[row_slot]).wait()

        def accum_chunk(row_slot, c):
            rv = rows_v.at[row_slot]
            seg = S // 4
            for r in range(CB):
                base = r * S
                zero = jnp.zeros((_L,), jnp.float32)

                @plsc.parallel_loop(0, seg, 1, unroll=8,
                                    carry=(zero,) * 8)
                def body(s, acc):
                    out = []
                    for g in range(4):
                        off = base + g * seg
                        a0 = acc[2 * g] + rv[off + s, 0:_L]
                        a1 = acc[2 * g + 1] + rv[off + s, _L:D]
                        out.extend((a0, a1))
                    return tuple(out)

                acc = body
                h0 = (acc[0] + acc[2]) + (acc[4] + acc[6])
                h1 = (acc[1] + acc[3]) + (acc[5] + acc[7])
                out_v[c * CB + r, 0:_L] = h0
                out_v[c * CB + r, _L:D] = h1

        for k in range(4):
            start_idx(k, k)
        for k in range(2):
            wait_idx(k)
            start_gather(k, k)

        def step(g4, _):
            for b in range(4):
                c = 4 * g4 + b
                rslot = b % 2
                wait_gather(b, rslot)
                accum_chunk(rslot, c)

                @pl.when(c + 2 < NCHUNK)
                def _():
                    wait_idx((b + 2) % 4)
                    start_gather((b + 2) % 4, rslot)

                @pl.when(c + 4 < NCHUNK)
                def _():
                    start_idx(c + 4, b)
            return 0

        lax.fori_loop(0, NCHUNK // 4, step, 0)
        pltpu.sync_copy(out_v, out_hbm.at[pl.ds(wid * b_per_w, b_per_w)])

    return kern


def _linear(B, S, D, C, BT=2048):
    scale = 1.0 / S

    def body(p_ref, w_ref, b_ref, o_ref):
        acc = jnp.dot(p_ref[...], w_ref[...],
                      preferred_element_type=jnp.float32)
        o_ref[...] = acc * scale + b_ref[...]

    return pl.pallas_call(
        body,
        grid=(B // BT,),
        in_specs=[
            pl.BlockSpec((BT, D), lambda i: (i, 0)),
            pl.BlockSpec((D, C), lambda i: (0, 0)),
            pl.BlockSpec((1, C), lambda i: (0, 0)),
        ],
        out_specs=pl.BlockSpec((BT, C), lambda i: (i, 0)),
        out_shape=jax.ShapeDtypeStruct((B, C), jnp.float32),
    )


def kernel(x, table, W, b):
    B, S = x.shape
    V, D = table.shape
    C = W.shape[1]
    x_flat = x.reshape(-1).astype(jnp.int32)
    sums = _gather_pool(B, S, D)(x_flat, table)
    return _linear(B, S, D, C)(sums, W, b.reshape(1, C))


# speedup vs baseline: 1.6526x; 1.4530x over previous
import functools

import jax
import jax.numpy as jnp
from jax import lax
from jax.experimental import layout as jex_layout
from jax.experimental import pallas as pl
from jax.experimental.pallas import tpu as pltpu
from jax.experimental.pallas import tpu_sc as plsc

_NC = 2
_NS = 16
_NW = _NC * _NS
_L = 16


def _gather_pool(B, S, D, CB=8):
    b_per_w = B // _NW
    ROWS = CB * S
    NCHUNK = b_per_w // CB
    assert B % _NW == 0 and b_per_w % CB == 0 and ROWS % 8 == 0
    assert D == 2 * _L and S % 4 == 0

    mesh = plsc.VectorSubcoreMesh(core_axis_name="c", subcore_axis_name="s",
                                  num_cores=_NC, num_subcores=_NS)

    @functools.partial(
        pl.kernel,
        out_type=jax.ShapeDtypeStruct((B, D), jnp.float32),
        mesh=mesh,
        scratch_types=[
            pltpu.VMEM((4 * ROWS,), jnp.int32),
            pltpu.VMEM((2, ROWS, D), jnp.float32),
            pltpu.VMEM((b_per_w, D), jnp.float32),
            pltpu.SemaphoreType.DMA,
            pltpu.SemaphoreType.DMA,
            pltpu.SemaphoreType.DMA,
            pltpu.SemaphoreType.DMA,
            pltpu.SemaphoreType.DMA,
            pltpu.SemaphoreType.DMA,
        ],
        compiler_params=pltpu.CompilerParams(use_tc_tiling_on_sc=False),
    )
    def kern(x_hbm, table_hbm, out_hbm, idx_v, rows_v, out_v,
             si0, si1, si2, si3, sr0, sr1):
        si = (si0, si1, si2, si3)
        sr = (sr0, sr1)
        wid = lax.axis_index("s") * _NC + lax.axis_index("c")
        xbase = wid * (b_per_w * S)

        def islot(slot):
            return idx_v.at[pl.ds(slot * ROWS, ROWS)]

        def start_idx(c, slot):
            pltpu.async_copy(
                x_hbm.at[pl.ds(xbase + c * ROWS, ROWS)], islot(slot),
                si[slot])

        def wait_idx(slot):
            pltpu.make_async_copy(
                x_hbm.at[pl.ds(0, ROWS)], islot(slot), si[slot]).wait()

        def start_gather(idx_slot, row_slot):
            pltpu.async_copy(
                table_hbm.at[islot(idx_slot)], rows_v.at[row_slot],
                sr[row_slot])

        def wait_gather(idx_slot, row_slot):
            pltpu.make_async_copy(
                table_hbm.at[islot(idx_slot)], rows_v.at[row_slot],
                sr[row_slot]).wait()

        def accum_chunk(row_slot, c):
            rv = rows_v.at[row_slot]
            seg = S // 4
            for r in range(CB):
                base = r * S
                zero = jnp.zeros((_L,), jnp.float32)

                @plsc.parallel_loop(0, seg, 1, unroll=8,
                                    carry=(zero,) * 8)
                def body(s, acc):
                    out = []
                    for g in range(4):
                        off = base + g * seg
                        a0 = acc[2 * g] + rv[off + s, 0:_L]
                        a1 = acc[2 * g + 1] + rv[off + s, _L:D]
                        out.extend((a0, a1))
                    return tuple(out)

                acc = body
                h0 = (acc[0] + acc[2]) + (acc[4] + acc[6])
                h1 = (acc[1] + acc[3]) + (acc[5] + acc[7])
                out_v[c * CB + r, 0:_L] = h0
                out_v[c * CB + r, _L:D] = h1

        for k in range(4):
            start_idx(k, k)
        for k in range(2):
            wait_idx(k)
            start_gather(k, k)

        def step(g4, _):
            for b in range(4):
                c = 4 * g4 + b
                rslot = b % 2
                wait_gather(b, rslot)
                accum_chunk(rslot, c)

                @pl.when(c + 2 < NCHUNK)
                def _():
                    wait_idx((b + 2) % 4)
                    start_gather((b + 2) % 4, rslot)

                @pl.when(c + 4 < NCHUNK)
                def _():
                    start_idx(c + 4, b)
            return 0

        lax.fori_loop(0, NCHUNK // 4, step, 0)
        pltpu.sync_copy(out_v, out_hbm.at[pl.ds(wid * b_per_w, b_per_w)])

    return kern


def _transpose_pack(V, D, BP=4096):
    VP = 1 << 20
    G = VP // 4
    NB = G // BP
    LAST = V // BP

    def body(t0, t1, t2, t3, o_ref):
        o_ref[...] = jnp.concatenate(
            [t[...].T for t in (t0, t1, t2, t3)], axis=1)

    def in_map(j):
        return lambda i: (0, jnp.minimum(j * NB + i, LAST))

    return pl.pallas_call(
        body,
        grid=(NB,),
        in_specs=[pl.BlockSpec((D, BP), in_map(j)) for j in range(4)],
        out_specs=pl.BlockSpec((BP, 4 * D), lambda i: (i, 0)),
        out_shape=jax.ShapeDtypeStruct((G, 4 * D), jnp.float32),
    )


def _linear(B, S, D, C, BT=2048):
    scale = 1.0 / S

    def body(p_ref, w_ref, b_ref, o_ref):
        acc = jnp.dot(p_ref[...], w_ref[...],
                      preferred_element_type=jnp.float32)
        o_ref[...] = acc * scale + b_ref[...]

    return pl.pallas_call(
        body,
        grid=(B // BT,),
        in_specs=[
            pl.BlockSpec((BT, D), lambda i: (i, 0)),
            pl.BlockSpec((D, C), lambda i: (0, 0)),
            pl.BlockSpec((1, C), lambda i: (0, 0)),
        ],
        out_specs=pl.BlockSpec((BT, C), lambda i: (i, 0)),
        out_shape=jax.ShapeDtypeStruct((B, C), jnp.float32),
    )


def kernel(x, table, W, b):
    B, S = x.shape
    V, D = table.shape
    C = W.shape[1]
    x_flat = x.reshape(-1).astype(jnp.int32)
    VP = 1 << 20
    tT = table.T
    packed = _transpose_pack(V, D)(tT, tT, tT, tT)
    table_rm = jex_layout.with_layout_constraint(
        packed.reshape(VP, D),
        jex_layout.Layout(major_to_minor=(0, 1), tiling=()))
    x_q = ((x_flat & (VP // 4 - 1)) << 2) | (x_flat >> (VP.bit_length() - 3))
    sums = _gather_pool(B, S, D)(x_q, table_rm)
    return _linear(B, S, D, C)(sums, W, b.reshape(1, C))


# speedup vs baseline: 2.1313x; 1.2897x over previous
import functools

import jax
import jax.numpy as jnp
from jax import lax
from jax.experimental import layout as jex_layout
from jax.experimental import pallas as pl
from jax.experimental.pallas import tpu as pltpu
from jax.experimental.pallas import tpu_sc as plsc

_NC = 2
_NS = 16
_NW = _NC * _NS
_L = 16


def _gather_pool(B, S, D, CB=8):
    b_per_w = B // _NW
    ROWS = CB * S
    NCHUNK = b_per_w // CB
    assert B % _NW == 0 and b_per_w % CB == 0 and ROWS % 8 == 0
    assert D == 2 * _L and S % 4 == 0

    mesh = plsc.VectorSubcoreMesh(core_axis_name="c", subcore_axis_name="s",
                                  num_cores=_NC, num_subcores=_NS)

    @functools.partial(
        pl.kernel,
        out_type=jax.ShapeDtypeStruct((B, D), jnp.float32),
        mesh=mesh,
        scratch_types=[
            pltpu.VMEM((4 * ROWS,), jnp.int32),
            pltpu.VMEM((2, ROWS, D), jnp.float32),
            pltpu.VMEM((b_per_w, D), jnp.float32),
            pltpu.SemaphoreType.DMA,
            pltpu.SemaphoreType.DMA,
            pltpu.SemaphoreType.DMA,
            pltpu.SemaphoreType.DMA,
            pltpu.SemaphoreType.DMA,
            pltpu.SemaphoreType.DMA,
        ],
        compiler_params=pltpu.CompilerParams(use_tc_tiling_on_sc=False),
    )
    def kern(x_hbm, table_hbm, out_hbm, idx_v, rows_v, out_v,
             si0, si1, si2, si3, sr0, sr1):
        si = (si0, si1, si2, si3)
        sr = (sr0, sr1)
        wid = lax.axis_index("s") * _NC + lax.axis_index("c")
        xbase = wid * (b_per_w * S)

        def islot(slot):
            return idx_v.at[pl.ds(slot * ROWS, ROWS)]

        def start_idx(c, slot):
            pltpu.async_copy(
                x_hbm.at[pl.ds(xbase + c * ROWS, ROWS)], islot(slot),
                si[slot])

        def wait_idx(slot):
            pltpu.make_async_copy(
                x_hbm.at[pl.ds(0, ROWS)], islot(slot), si[slot]).wait()

        def start_gather(idx_slot, row_slot):
            pltpu.async_copy(
                table_hbm.at[islot(idx_slot)], rows_v.at[row_slot],
                sr[row_slot])

        def wait_gather(idx_slot, row_slot):
            pltpu.make_async_copy(
                table_hbm.at[islot(idx_slot)], rows_v.at[row_slot],
                sr[row_slot]).wait()

        def accum_chunk(row_slot, c):
            rv = rows_v.at[row_slot]
            seg = S // 4
            for r in range(CB):
                base = r * S
                zero = jnp.zeros((_L,), jnp.float32)

                @plsc.parallel_loop(0, seg, 1, unroll=8,
                                    carry=(zero,) * 8)
                def body(s, acc):
                    out = []
                    for g in range(4):
                        off = base + g * seg
                        a0 = acc[2 * g] + rv[off + s, 0:_L]
                        a1 = acc[2 * g + 1] + rv[off + s, _L:D]
                        out.extend((a0, a1))
                    return tuple(out)

                acc = body
                h0 = (acc[0] + acc[2]) + (acc[4] + acc[6])
                h1 = (acc[1] + acc[3]) + (acc[5] + acc[7])
                out_v[c * CB + r, 0:_L] = h0
                out_v[c * CB + r, _L:D] = h1

        for k in range(4):
            start_idx(k, k)
        for k in range(2):
            wait_idx(k)
            start_gather(k, k)

        def step(g4, _):
            for b in range(4):
                c = 4 * g4 + b
                rslot = b % 2
                wait_gather(b, rslot)
                accum_chunk(rslot, c)

                @pl.when(c + 2 < NCHUNK)
                def _():
                    wait_idx((b + 2) % 4)
                    start_gather((b + 2) % 4, rslot)

                @pl.when(c + 4 < NCHUNK)
                def _():
                    start_idx(c + 4, b)
            return 0

        lax.fori_loop(0, NCHUNK // 4, step, 0)
        pltpu.sync_copy(out_v, out_hbm.at[pl.ds(wid * b_per_w, b_per_w)])

    return kern


def _transpose_pack(V, D, BP=2048):
    VP = 1 << 20
    G = VP // 4
    NB = G // BP
    LAST = V // BP

    def body(t0, t1, t2, t3, o_ref):
        o_ref[...] = jnp.concatenate(
            [t[...] for t in (t0, t1, t2, t3)], axis=0).T

    def in_map(j):
        return lambda i: (0, jnp.minimum(j * NB + i, LAST))

    return pl.pallas_call(
        body,
        grid=(NB,),
        in_specs=[pl.BlockSpec((D, BP), in_map(j)) for j in range(4)],
        out_specs=pl.BlockSpec((BP, 4 * D), lambda i: (i, 0)),
        out_shape=jax.ShapeDtypeStruct((G, 4 * D), jnp.float32),
    )


def _linear(B, S, D, C, BT=2048):
    scale = 1.0 / S

    def body(p_ref, w_ref, b_ref, o_ref):
        acc = jnp.dot(p_ref[...], w_ref[...],
                      preferred_element_type=jnp.float32)
        o_ref[...] = acc * scale + b_ref[...]

    return pl.pallas_call(
        body,
        grid=(B // BT,),
        in_specs=[
            pl.BlockSpec((BT, D), lambda i: (i, 0)),
            pl.BlockSpec((D, C), lambda i: (0, 0)),
            pl.BlockSpec((1, C), lambda i: (0, 0)),
        ],
        out_specs=pl.BlockSpec((BT, C), lambda i: (i, 0)),
        out_shape=jax.ShapeDtypeStruct((B, C), jnp.float32),
    )


def kernel(x, table, W, b):
    B, S = x.shape
    V, D = table.shape
    C = W.shape[1]
    x_flat = x.reshape(-1).astype(jnp.int32)
    VP = 1 << 20
    tT = table.T
    packed = _transpose_pack(V, D)(tT, tT, tT, tT)
    table_rm = jex_layout.with_layout_constraint(
        packed.reshape(VP, D),
        jex_layout.Layout(major_to_minor=(0, 1), tiling=()))
    x_q = ((x_flat & (VP // 4 - 1)) << 2) | (x_flat >> (VP.bit_length() - 3))
    sums = _gather_pool(B, S, D)(x_q, table_rm)
    return _linear(B, S, D, C)(sums, W, b.reshape(1, C))


# speedup vs baseline: 2.2072x; 1.0356x over previous
import functools

import jax
import jax.numpy as jnp
from jax import lax
from jax.experimental import layout as jex_layout
from jax.experimental import pallas as pl
from jax.experimental.pallas import tpu as pltpu
from jax.experimental.pallas import tpu_sc as plsc

_NC = 2
_NS = 16
_NW = _NC * _NS
_L = 16


def _gather_pool(B, S, D, CB=4):
    b_per_w = B // _NW
    ROWS = CB * S
    NCHUNK = b_per_w // CB
    assert B % _NW == 0 and b_per_w % CB == 0 and ROWS % 8 == 0
    assert D == 2 * _L and S % 4 == 0

    mesh = plsc.VectorSubcoreMesh(core_axis_name="c", subcore_axis_name="s",
                                  num_cores=_NC, num_subcores=_NS)

    @functools.partial(
        pl.kernel,
        out_type=jax.ShapeDtypeStruct((B, D), jnp.float32),
        mesh=mesh,
        scratch_types=[
            pltpu.VMEM((8 * ROWS,), jnp.int32),
            pltpu.VMEM((4, ROWS, D), jnp.float32),
            pltpu.VMEM((b_per_w, D), jnp.float32),
            pltpu.SemaphoreType.DMA,
            pltpu.SemaphoreType.DMA,
            pltpu.SemaphoreType.DMA,
            pltpu.SemaphoreType.DMA,
            pltpu.SemaphoreType.DMA,
            pltpu.SemaphoreType.DMA,
            pltpu.SemaphoreType.DMA,
            pltpu.SemaphoreType.DMA,
            pltpu.SemaphoreType.DMA,
            pltpu.SemaphoreType.DMA,
            pltpu.SemaphoreType.DMA,
            pltpu.SemaphoreType.DMA,
        ],
        compiler_params=pltpu.CompilerParams(use_tc_tiling_on_sc=False),
    )
    def kern(x_hbm, table_hbm, out_hbm, idx_v, rows_v, out_v,
             si0, si1, si2, si3, si4, si5, si6, si7, sr0, sr1, sr2, sr3):
        si = (si0, si1, si2, si3, si4, si5, si6, si7)
        sr = (sr0, sr1, sr2, sr3)
        wid = lax.axis_index("s") * _NC + lax.axis_index("c")
        xbase = wid * (b_per_w * S)

        def islot(slot):
            return idx_v.at[pl.ds(slot * ROWS, ROWS)]

        def start_idx(c, slot):
            pltpu.async_copy(
                x_hbm.at[pl.ds(xbase + c * ROWS, ROWS)], islot(slot),
                si[slot])

        def wait_idx(slot):
            pltpu.make_async_copy(
                x_hbm.at[pl.ds(0, ROWS)], islot(slot), si[slot]).wait()

        def start_gather(idx_slot, row_slot):
            pltpu.async_copy(
                table_hbm.at[islot(idx_slot)], rows_v.at[row_slot],
                sr[row_slot])

        def wait_gather(idx_slot, row_slot):
            pltpu.make_async_copy(
                table_hbm.at[islot(idx_slot)], rows_v.at[row_slot],
                sr[row_slot]).wait()

        def accum_chunk(row_slot, c):
            rv = rows_v.at[row_slot]
            seg = S // 4
            for r in range(CB):
                base = r * S
                zero = jnp.zeros((_L,), jnp.float32)

                @plsc.parallel_loop(0, seg, 1, unroll=8,
                                    carry=(zero,) * 8)
                def body(s, acc):
                    out = []
                    for g in range(4):
                        off = base + g * seg
                        a0 = acc[2 * g] + rv[off + s, 0:_L]
                        a1 = acc[2 * g + 1] + rv[off + s, _L:D]
                        out.extend((a0, a1))
                    return tuple(out)

                acc = body
                h0 = (acc[0] + acc[2]) + (acc[4] + acc[6])
                h1 = (acc[1] + acc[3]) + (acc[5] + acc[7])
                out_v[c * CB + r, 0:_L] = h0
                out_v[c * CB + r, _L:D] = h1

        for k in range(8):
            start_idx(k, k)
        for k in range(3):
            wait_idx(k)
            start_gather(k, k)

        def step(g8, _):
            for b in range(8):
                c = 8 * g8 + b
                rslot = b % 4
                wait_gather(b, rslot)
                accum_chunk(rslot, c)

                @pl.when(c + 3 < NCHUNK)
                def _():
                    wait_idx((b + 3) % 8)
                    start_gather((b + 3) % 8, (b + 3) % 4)

                @pl.when(c + 8 < NCHUNK)
                def _():
                    start_idx(c + 8, b)
            return 0

        lax.fori_loop(0, NCHUNK // 8, step, 0)
        pltpu.sync_copy(out_v, out_hbm.at[pl.ds(wid * b_per_w, b_per_w)])

    return kern


def _transpose_pack(V, D, BP=2048):
    VP = 1 << 20
    G = VP // 4
    NB = G // BP
    LAST = V // BP

    def body(t0, t1, t2, t3, o_ref):
        o_ref[...] = jnp.concatenate(
            [t[...] for t in (t0, t1, t2, t3)], axis=0).T

    def in_map(j):
        return lambda i: (0, jnp.minimum(j * NB + i, LAST))

    return pl.pallas_call(
        body,
        grid=(NB,),
        in_specs=[pl.BlockSpec((D, BP), in_map(j)) for j in range(4)],
        out_specs=pl.BlockSpec((BP, 4 * D), lambda i: (i, 0)),
        out_shape=jax.ShapeDtypeStruct((G, 4 * D), jnp.float32),
    )


def _linear(B, S, D, C, BT=2048):
    scale = 1.0 / S

    def body(p_ref, w_ref, b_ref, o_ref):
        acc = jnp.dot(p_ref[...], w_ref[...],
                      preferred_element_type=jnp.float32)
        o_ref[...] = acc * scale + b_ref[...]

    return pl.pallas_call(
        body,
        grid=(B // BT,),
        in_specs=[
            pl.BlockSpec((BT, D), lambda i: (i, 0)),
            pl.BlockSpec((D, C), lambda i: (0, 0)),
            pl.BlockSpec((1, C), lambda i: (0, 0)),
        ],
        out_specs=pl.BlockSpec((BT, C), lambda i: (i, 0)),
        out_shape=jax.ShapeDtypeStruct((B, C), jnp.float32),
    )


def kernel(x, table, W, b):
    B, S = x.shape
    V, D = table.shape
    C = W.shape[1]
    x_flat = x.reshape(-1).astype(jnp.int32)
    VP = 1 << 20
    tT = table.T
    packed = _transpose_pack(V, D)(tT, tT, tT, tT)
    table_rm = jex_layout.with_layout_constraint(
        packed.reshape(VP, D),
        jex_layout.Layout(major_to_minor=(0, 1), tiling=()))
    x_q = ((x_flat & (VP // 4 - 1)) << 2) | (x_flat >> (VP.bit_length() - 3))
    sums = _gather_pool(B, S, D)(x_q, table_rm)
    return _linear(B, S, D, C)(sums, W, b.reshape(1, C))


# speedup vs baseline: 2.2096x; 1.0011x over previous
import functools

import jax
import jax.numpy as jnp
from jax import lax
from jax.experimental import layout as jex_layout
from jax.experimental import pallas as pl
from jax.experimental.pallas import tpu as pltpu
from jax.experimental.pallas import tpu_sc as plsc

_NC = 2
_NS = 16
_NW = _NC * _NS
_L = 16


def _gather_pool(B, S, D, CB=4):
    b_per_w = B // _NW
    ROWS = CB * S
    NCHUNK = b_per_w // CB
    assert B % _NW == 0 and b_per_w % CB == 0 and ROWS % 8 == 0
    assert D == 2 * _L and S % 4 == 0

    mesh = plsc.VectorSubcoreMesh(core_axis_name="c", subcore_axis_name="s",
                                  num_cores=_NC, num_subcores=_NS)

    @functools.partial(
        pl.kernel,
        out_type=jax.ShapeDtypeStruct((B, D), jnp.float32),
        mesh=mesh,
        scratch_types=[
            pltpu.VMEM((8 * ROWS,), jnp.int32),
            pltpu.VMEM((4, ROWS, D), jnp.float32),
            pltpu.VMEM((b_per_w, D), jnp.float32),
            pltpu.SemaphoreType.DMA,
            pltpu.SemaphoreType.DMA,
            pltpu.SemaphoreType.DMA,
            pltpu.SemaphoreType.DMA,
            pltpu.SemaphoreType.DMA,
            pltpu.SemaphoreType.DMA,
            pltpu.SemaphoreType.DMA,
            pltpu.SemaphoreType.DMA,
            pltpu.SemaphoreType.DMA,
            pltpu.SemaphoreType.DMA,
            pltpu.SemaphoreType.DMA,
            pltpu.SemaphoreType.DMA,
        ],
        compiler_params=pltpu.CompilerParams(use_tc_tiling_on_sc=False),
    )
    def kern(x_hbm, table_hbm, out_hbm, idx_v, rows_v, out_v,
             si0, si1, si2, si3, si4, si5, si6, si7, sr0, sr1, sr2, sr3):
        si = (si0, si1, si2, si3, si4, si5, si6, si7)
        sr = (sr0, sr1, sr2, sr3)
        wid = lax.axis_index("s") * _NC + lax.axis_index("c")
        xbase = wid * (b_per_w * S)

        def islot(slot):
            return idx_v.at[pl.ds(slot * ROWS, ROWS)]

        def start_idx(c, slot):
            pltpu.async_copy(
                x_hbm.at[pl.ds(xbase + c * ROWS, ROWS)], islot(slot),
                si[slot])

        def wait_idx(slot):
            pltpu.make_async_copy(
                x_hbm.at[pl.ds(0, ROWS)], islot(slot), si[slot]).wait()

        def start_gather(idx_slot, row_slot):
            pltpu.async_copy(
                table_hbm.at[islot(idx_slot)], rows_v.at[row_slot],
                sr[row_slot])

        def wait_gather(idx_slot, row_slot):
            pltpu.make_async_copy(
                table_hbm.at[islot(idx_slot)], rows_v.at[row_slot],
                sr[row_slot]).wait()

        def accum_chunk(row_slot, c):
            rv = rows_v.at[row_slot]
            seg = S // 4
            for r in range(CB):
                base = r * S
                zero = jnp.zeros((_L,), jnp.float32)

                @plsc.parallel_loop(0, seg, 1, unroll=8,
                                    carry=(zero,) * 8)
                def body(s, acc):
                    out = []
                    for g in range(4):
                        off = base + g * seg
                        a0 = acc[2 * g] + rv[off + s, 0:_L]
                        a1 = acc[2 * g + 1] + rv[off + s, _L:D]
                        out.extend((a0, a1))
                    return tuple(out)

                acc = body
                h0 = (acc[0] + acc[2]) + (acc[4] + acc[6])
                h1 = (acc[1] + acc[3]) + (acc[5] + acc[7])
                out_v[c * CB + r, 0:_L] = h0
                out_v[c * CB + r, _L:D] = h1

        for k in range(8):
            start_idx(k, k)
        for k in range(3):
            wait_idx(k)
            start_gather(k, k)

        def step(g8, _):
            for b in range(8):
                c = 8 * g8 + b
                rslot = b % 4
                wait_gather(b, rslot)

                @pl.when(c + 3 < NCHUNK)
                def _():
                    wait_idx((b + 3) % 8)
                    start_gather((b + 3) % 8, (b + 3) % 4)

                @pl.when(c + 8 < NCHUNK)
                def _():
                    start_idx(c + 8, b)

                accum_chunk(rslot, c)
            return 0

        lax.fori_loop(0, NCHUNK // 8, step, 0)
        pltpu.sync_copy(out_v, out_hbm.at[pl.ds(wid * b_per_w, b_per_w)])

    return kern


def _transpose_pack(V, D, BP=2048):
    VP = 1 << 20
    G = VP // 4
    NB = G // BP
    LAST = V // BP

    def body(t0, t1, t2, t3, o_ref):
        o_ref[...] = jnp.concatenate(
            [t[...] for t in (t0, t1, t2, t3)], axis=0).T

    def in_map(j):
        return lambda i: (0, jnp.minimum(j * NB + i, LAST))

    return pl.pallas_call(
        body,
        grid=(NB,),
        in_specs=[pl.BlockSpec((D, BP), in_map(j)) for j in range(4)],
        out_specs=pl.BlockSpec((BP, 4 * D), lambda i: (i, 0)),
        out_shape=jax.ShapeDtypeStruct((G, 4 * D), jnp.float32),
    )


def _linear(B, S, D, C, BT=2048):
    scale = 1.0 / S

    def body(p_ref, w_ref, b_ref, o_ref):
        acc = jnp.dot(p_ref[...], w_ref[...],
                      preferred_element_type=jnp.float32)
        o_ref[...] = acc * scale + b_ref[...]

    return pl.pallas_call(
        body,
        grid=(B // BT,),
        in_specs=[
            pl.BlockSpec((BT, D), lambda i: (i, 0)),
            pl.BlockSpec((D, C), lambda i: (0, 0)),
            pl.BlockSpec((1, C), lambda i: (0, 0)),
        ],
        out_specs=pl.BlockSpec((BT, C), lambda i: (i, 0)),
        out_shape=jax.ShapeDtypeStruct((B, C), jnp.float32),
    )


def kernel(x, table, W, b):
    B, S = x.shape
    V, D = table.shape
    C = W.shape[1]
    x_flat = x.reshape(-1).astype(jnp.int32)
    VP = 1 << 20
    tT = table.T
    packed = _transpose_pack(V, D)(tT, tT, tT, tT)
    table_rm = jex_layout.with_layout_constraint(
        packed.reshape(VP, D),
        jex_layout.Layout(major_to_minor=(0, 1), tiling=()))
    x_q = ((x_flat & (VP // 4 - 1)) << 2) | (x_flat >> (VP.bit_length() - 3))
    sums = _gather_pool(B, S, D)(x_q, table_rm)
    return _linear(B, S, D, C)(sums, W, b.reshape(1, C))
